# baseline (speedup 1.0000x reference)
import functools

import jax
import jax.numpy as jnp
from jax import lax
from jax.experimental import pallas as pl
from jax.experimental.pallas import tpu as pltpu
from jax.experimental.pallas import tpu_sc as plsc

_M = 8192
_D = 768
_NC = 2
_NS = 16
_NW = _NC * _NS
_ROWS_PER_W = _M // _NW
_CHUNK = 128


def _make_sc_copy():
    mesh = plsc.VectorSubcoreMesh(core_axis_name="c", subcore_axis_name="s")

    @functools.partial(
        pl.kernel,
        mesh=mesh,
        out_type=jax.ShapeDtypeStruct((_M, _D), jnp.float32),
        scratch_types=[pltpu.VMEM((_CHUNK, _D), jnp.float32)],
    )
    def sc_copy(src_hbm, out_hbm, buf):
        wid = lax.axis_index("s") * _NC + lax.axis_index("c")
        base = wid * _ROWS_PER_W
        for j in range(_ROWS_PER_W // _CHUNK):
            off = base + j * _CHUNK
            pltpu.sync_copy(src_hbm.at[pl.ds(off, _CHUNK), :], buf)
            pltpu.sync_copy(buf, out_hbm.at[pl.ds(off, _CHUNK), :])

    return sc_copy


_sc_copy = _make_sc_copy()


def kernel(x, pos_emb):
    del x
    return _sc_copy(pos_emb)
